# baseline (speedup 1.0000x reference)
import functools

import jax
import jax.numpy as jnp
from jax import lax
from jax.experimental import pallas as pl
from jax.experimental.pallas import tpu as pltpu
from jax.experimental.pallas import tpu_sc as plsc

NC = 2
NS = 16
NW = NC * NS
L = 16
D = 64
B = 16384
B_PER_W = B // NW
CT = 64
NCH = B_PER_W // CT


def _sc_gather_body(tgt_idx, anc_idx, tgt_tab, anc_tab,
                    tgt_out, anc_out,
                    idx_t_v, idx_a_v, rows_v, sem, osem):
    wid = lax.axis_index("s") * NC + lax.axis_index("c")
    base = wid * B_PER_W
    pltpu.sync_copy(tgt_idx.at[pl.ds(base, B_PER_W)], idx_t_v)
    pltpu.sync_copy(anc_idx.at[pl.ds(base, B_PER_W)], idx_a_v)
    def table(tab, idx_v, out):
        copies = []
        for j in range(B_PER_W // 128):
            copies.append(pltpu.async_copy(
                tab.at[idx_v.at[pl.ds(j * 128, 128)]],
                rows_v.at[pl.ds(j * 128, 128)], sem))
        for c in copies:
            c.wait()
        pltpu.sync_copy(rows_v, out.at[pl.ds(base, B_PER_W)])

    table(tgt_tab, idx_t_v, tgt_out)
    table(anc_tab, idx_a_v, anc_out)


@jax.jit
def _sc_gather(tgt_idx, anc_idx, tgt_tab, anc_tab):
    mesh = plsc.VectorSubcoreMesh(
        core_axis_name="c", subcore_axis_name="s",
        num_cores=NC, num_subcores=NS)
    return pl.kernel(
        _sc_gather_body,
        out_type=[
            jax.ShapeDtypeStruct((B, 128), jnp.float32),
            jax.ShapeDtypeStruct((B, 128), jnp.float32),
        ],
        mesh=mesh,
        scratch_types=[
            pltpu.VMEM((B_PER_W,), jnp.int32),
            pltpu.VMEM((B_PER_W,), jnp.int32),
            pltpu.VMEM((B_PER_W, 128), jnp.float32),
            pltpu.SemaphoreType.DMA,
            pltpu.SemaphoreType.DMA,
        ],
        compiler_params=pltpu.CompilerParams(
            use_tc_tiling_on_sc=False),
    )(tgt_idx, anc_idx, tgt_tab, anc_tab)


BLK = 2048


def _score_body(r0_ref, r1_ref, t_ref, a_ref, o_ref):
    R = jnp.dot(r0_ref[...], r1_ref[...], preferred_element_type=jnp.float32)
    P = jnp.dot(a_ref[...][:, :D], R, preferred_element_type=jnp.float32)
    T = t_ref[...][:, :D]
    num = jnp.sum(P * T, axis=1)
    den2 = jnp.sum(P * P, axis=1) * jnp.sum(T * T, axis=1)
    o_ref[...] = num / jnp.maximum(jnp.sqrt(den2), 1e-12)


@jax.jit
def _tc_score(rel_mat0, rel_mat1, t_rows, a_rows):
    out = pl.pallas_call(
        _score_body,
        grid=(B // BLK,),
        in_specs=[
            pl.BlockSpec((D, D), lambda i: (0, 0)),
            pl.BlockSpec((D, D), lambda i: (0, 0)),
            pl.BlockSpec((BLK, 128), lambda i: (i, 0)),
            pl.BlockSpec((BLK, 128), lambda i: (i, 0)),
        ],
        out_specs=pl.BlockSpec((BLK,), lambda i: (i,)),
        out_shape=jax.ShapeDtypeStruct((B,), jnp.float32),
    )(rel_mat0, rel_mat1, t_rows, a_rows)
    return out


def kernel(target_nodes, anchor_nodes, target_table, anchor_table, rel_mat0, rel_mat1):
    tgt_idx = target_nodes.astype(jnp.int32)
    anc_idx = anchor_nodes.astype(jnp.int32)
    t_pad = jnp.pad(target_table, ((0, 0), (0, 128 - D)))
    a_pad = jnp.pad(anchor_table, ((0, 0), (0, 128 - D)))
    t_rows, a_rows = _sc_gather(tgt_idx, anc_idx, t_pad, a_pad)
    return _tc_score(rel_mat0, rel_mat1, t_rows, a_rows)


# speedup vs baseline: 1.2025x; 1.2025x over previous
import functools

import jax
import jax.numpy as jnp
from jax import lax
from jax.experimental import pallas as pl
from jax.experimental.pallas import tpu as pltpu
from jax.experimental.pallas import tpu_sc as plsc

NC = 2
NS = 16
NW = NC * NS
L = 16
D = 64
B = 16384
B_PER_W = B // NW
CT = 64
NCH = B_PER_W // CT


def _sc_gather_body(tgt_idx, anc_idx, tgt_tab, anc_tab,
                    tgt_out, anc_out,
                    idx_t_v, idx_a_v, rows_t_v, rows_a_v, sem, osem):
    wid = lax.axis_index("s") * NC + lax.axis_index("c")
    base = wid * B_PER_W
    pltpu.sync_copy(tgt_idx.at[pl.ds(base, B_PER_W)], idx_t_v)
    pltpu.sync_copy(anc_idx.at[pl.ds(base, B_PER_W)], idx_a_v)
    lanes = jax.lax.iota(jnp.int32, L)

    def fire(c, tab, idx_v, rows_v):
        copies = []
        for g in range(CT // L):
            vec = idx_v[pl.ds(c * CT + g * L, L)]
            for j in range(L):
                row = jnp.sum(jnp.where(lanes == j, vec, 0))
                copies.append(pltpu.async_copy(
                    tab.at[pl.ds(row, 1)],
                    rows_v.at[pl.ds(g * L + j, 1)], sem))
        return copies

    def body(c, carry):
        copies = fire(c, tgt_tab, idx_t_v, rows_t_v)
        copies += fire(c, anc_tab, idx_a_v, rows_a_v)
        for cp in copies:
            cp.wait()
        pltpu.async_copy(
            rows_t_v, tgt_out.at[pl.ds(base + c * CT, CT)], osem).wait()
        pltpu.async_copy(
            rows_a_v, anc_out.at[pl.ds(base + c * CT, CT)], osem).wait()
        return carry

    lax.fori_loop(0, NCH, body, 0)


@jax.jit
def _sc_gather(tgt_idx, anc_idx, tgt_tab, anc_tab):
    mesh = plsc.VectorSubcoreMesh(
        core_axis_name="c", subcore_axis_name="s",
        num_cores=NC, num_subcores=NS)
    return pl.kernel(
        _sc_gather_body,
        out_type=[
            jax.ShapeDtypeStruct((B, D), jnp.float32),
            jax.ShapeDtypeStruct((B, D), jnp.float32),
        ],
        mesh=mesh,
        scratch_types=[
            pltpu.VMEM((B_PER_W,), jnp.int32),
            pltpu.VMEM((B_PER_W,), jnp.int32),
            pltpu.VMEM((CT, D), jnp.float32),
            pltpu.VMEM((CT, D), jnp.float32),
            pltpu.SemaphoreType.DMA,
            pltpu.SemaphoreType.DMA,
        ],
        compiler_params=pltpu.CompilerParams(
            use_tc_tiling_on_sc=True, needs_layout_passes=False),
    )(tgt_idx, anc_idx, tgt_tab, anc_tab)


BLK = 2048


def _score_body(r0_ref, r1_ref, t_ref, a_ref, o_ref):
    R = jnp.dot(r0_ref[...], r1_ref[...], preferred_element_type=jnp.float32)
    P = jnp.dot(a_ref[...], R, preferred_element_type=jnp.float32)
    T = t_ref[...]
    num = jnp.sum(P * T, axis=1)
    den2 = jnp.sum(P * P, axis=1) * jnp.sum(T * T, axis=1)
    o_ref[...] = num / jnp.maximum(jnp.sqrt(den2), 1e-12)


@jax.jit
def _tc_score(rel_mat0, rel_mat1, t_rows, a_rows):
    out = pl.pallas_call(
        _score_body,
        grid=(B // BLK,),
        in_specs=[
            pl.BlockSpec((D, D), lambda i: (0, 0)),
            pl.BlockSpec((D, D), lambda i: (0, 0)),
            pl.BlockSpec((BLK, D), lambda i: (i, 0)),
            pl.BlockSpec((BLK, D), lambda i: (i, 0)),
        ],
        out_specs=pl.BlockSpec((BLK,), lambda i: (i,)),
        out_shape=jax.ShapeDtypeStruct((B,), jnp.float32),
    )(rel_mat0, rel_mat1, t_rows, a_rows)
    return out


def kernel(target_nodes, anchor_nodes, target_table, anchor_table, rel_mat0, rel_mat1):
    tgt_idx = target_nodes.astype(jnp.int32)
    anc_idx = anchor_nodes.astype(jnp.int32)
    t_rows, a_rows = _sc_gather(tgt_idx, anc_idx, target_table, anchor_table)
    return _tc_score(rel_mat0, rel_mat1, t_rows, a_rows)


# speedup vs baseline: 1.2674x; 1.0540x over previous
import functools

import jax
import jax.numpy as jnp
from jax import lax
from jax.experimental import pallas as pl
from jax.experimental.pallas import tpu as pltpu
from jax.experimental.pallas import tpu_sc as plsc

NC = 2
NS = 16
NW = NC * NS
L = 16
D = 64
B = 16384
B_PER_W = B // NW
CT = 64
NCH = B_PER_W // CT


def _sc_gather_body(idx, tab, out, idx_v, rows_a, rows_b, sem, osem):
    wid = lax.axis_index("s") * NC + lax.axis_index("c")
    base = wid * B_PER_W
    pltpu.sync_copy(idx.at[pl.ds(base, B_PER_W)], idx_v)
    lanes = jax.lax.iota(jnp.int32, L)

    def fire(c, rows_v):
        copies = []
        for g in range(CT // L):
            vec = idx_v[pl.ds(c * CT + g * L, L)]
            for j in range(L):
                row = jnp.sum(jnp.where(lanes == j, vec, 0))
                copies.append(pltpu.async_copy(
                    tab.at[pl.ds(row, 1)],
                    rows_v.at[pl.ds(g * L + j, 1)], sem))
        return copies

    def body(c, carry):
        copies = fire(2 * c, rows_a)
        copies += fire(2 * c + 1, rows_b)
        for cp in copies:
            cp.wait()
        pltpu.async_copy(
            rows_a, out.at[pl.ds(base + 2 * c * CT, CT)], osem).wait()
        pltpu.async_copy(
            rows_b, out.at[pl.ds(base + (2 * c + 1) * CT, CT)], osem).wait()
        return carry

    lax.fori_loop(0, NCH // 2, body, 0)


def _sc_gather_one(idx, tab):
    mesh = plsc.VectorSubcoreMesh(
        core_axis_name="c", subcore_axis_name="s",
        num_cores=NC, num_subcores=NS)
    return pl.kernel(
        _sc_gather_body,
        out_type=jax.ShapeDtypeStruct((B, D), jnp.float32),
        mesh=mesh,
        scratch_types=[
            pltpu.VMEM((B_PER_W,), jnp.int32),
            pltpu.VMEM((CT, D), jnp.float32),
            pltpu.VMEM((CT, D), jnp.float32),
            pltpu.SemaphoreType.DMA,
            pltpu.SemaphoreType.DMA,
        ],
        compiler_params=pltpu.CompilerParams(
            use_tc_tiling_on_sc=True, needs_layout_passes=False),
    )(idx, tab)


@jax.jit
def _sc_gather(tgt_idx, anc_idx, tgt_tab, anc_tab):
    return (_sc_gather_one(tgt_idx, tgt_tab),
            _sc_gather_one(anc_idx, anc_tab))


BLK = 2048


def _score_body(r0_ref, r1_ref, t_ref, a_ref, o_ref):
    R = jnp.dot(r0_ref[...], r1_ref[...], preferred_element_type=jnp.float32)
    P = jnp.dot(a_ref[...], R, preferred_element_type=jnp.float32)
    T = t_ref[...]
    num = jnp.sum(P * T, axis=1)
    den2 = jnp.sum(P * P, axis=1) * jnp.sum(T * T, axis=1)
    o_ref[...] = num / jnp.maximum(jnp.sqrt(den2), 1e-12)


@jax.jit
def _tc_score(rel_mat0, rel_mat1, t_rows, a_rows):
    out = pl.pallas_call(
        _score_body,
        grid=(B // BLK,),
        in_specs=[
            pl.BlockSpec((D, D), lambda i: (0, 0)),
            pl.BlockSpec((D, D), lambda i: (0, 0)),
            pl.BlockSpec((BLK, D), lambda i: (i, 0)),
            pl.BlockSpec((BLK, D), lambda i: (i, 0)),
        ],
        out_specs=pl.BlockSpec((BLK,), lambda i: (i,)),
        out_shape=jax.ShapeDtypeStruct((B,), jnp.float32),
    )(rel_mat0, rel_mat1, t_rows, a_rows)
    return out


def kernel(target_nodes, anchor_nodes, target_table, anchor_table, rel_mat0, rel_mat1):
    tgt_idx = target_nodes.astype(jnp.int32)
    anc_idx = anchor_nodes.astype(jnp.int32)
    t_rows, a_rows = _sc_gather(tgt_idx, anc_idx, target_table, anchor_table)
    return _tc_score(rel_mat0, rel_mat1, t_rows, a_rows)


# speedup vs baseline: 1.2720x; 1.0036x over previous
import functools

import jax
import jax.numpy as jnp
from jax import lax
from jax.experimental import pallas as pl
from jax.experimental.pallas import tpu as pltpu
from jax.experimental.pallas import tpu_sc as plsc

NC = 2
NS = 16
NW = NC * NS
L = 16
D = 64
B = 16384
B_PER_W = B // NW
CT = 64
NCH = B_PER_W // CT


def _sc_gather_body(idx, tab, out, idx_v, rows_a, rows_b, sem, osem):
    wid = lax.axis_index("s") * NC + lax.axis_index("c")
    base = wid * B_PER_W
    pltpu.sync_copy(idx.at[pl.ds(base, B_PER_W)], idx_v)
    lanes = jax.lax.iota(jnp.int32, L)

    def fire(c, rows_v):
        copies = []
        for g in range(CT // L):
            vec = idx_v[pl.ds(c * CT + g * L, L)]
            for j in range(L):
                row = jnp.sum(jnp.where(lanes == j, vec, 0))
                copies.append(pltpu.async_copy(
                    tab.at[pl.ds(row, 1)],
                    rows_v.at[pl.ds(g * L + j, 1)], sem))
        return copies

    def body(c, carry):
        copies = fire(2 * c, rows_a)
        copies += fire(2 * c + 1, rows_b)
        for cp in copies:
            cp.wait()
        pltpu.async_copy(
            rows_a, out.at[pl.ds(base + 2 * c * CT, CT)], osem).wait()
        pltpu.async_copy(
            rows_b, out.at[pl.ds(base + (2 * c + 1) * CT, CT)], osem).wait()
        return carry

    lax.fori_loop(0, NCH // 2, body, 0)


def _sc_gather_one(idx, tab):
    mesh = plsc.VectorSubcoreMesh(
        core_axis_name="c", subcore_axis_name="s",
        num_cores=NC, num_subcores=NS)
    return pl.kernel(
        _sc_gather_body,
        out_type=jax.ShapeDtypeStruct((B, D), jnp.float32),
        mesh=mesh,
        scratch_types=[
            pltpu.VMEM((B_PER_W,), jnp.int32),
            pltpu.VMEM((CT, D), jnp.float32),
            pltpu.VMEM((CT, D), jnp.float32),
            pltpu.SemaphoreType.DMA,
            pltpu.SemaphoreType.DMA,
        ],
        compiler_params=pltpu.CompilerParams(
            use_tc_tiling_on_sc=True, needs_layout_passes=False),
    )(idx, tab)


@jax.jit
def _sc_gather(tgt_idx, anc_idx, tgt_tab, anc_tab):
    return (_sc_gather_one(tgt_idx, tgt_tab),
            _sc_gather_one(anc_idx, anc_tab))


BLK = 4096


def _score_body(r0_ref, r1_ref, t_ref, a_ref, o_ref):
    R = jnp.dot(r0_ref[...], r1_ref[...], preferred_element_type=jnp.float32)
    P = jnp.dot(a_ref[...], R, preferred_element_type=jnp.float32)
    T = t_ref[...]
    num = jnp.sum(P * T, axis=1)
    den2 = jnp.sum(P * P, axis=1) * jnp.sum(T * T, axis=1)
    o_ref[...] = num / jnp.maximum(jnp.sqrt(den2), 1e-12)


@jax.jit
def _tc_score(rel_mat0, rel_mat1, t_rows, a_rows):
    out = pl.pallas_call(
        _score_body,
        grid=(B // BLK,),
        in_specs=[
            pl.BlockSpec((D, D), lambda i: (0, 0)),
            pl.BlockSpec((D, D), lambda i: (0, 0)),
            pl.BlockSpec((BLK, D), lambda i: (i, 0)),
            pl.BlockSpec((BLK, D), lambda i: (i, 0)),
        ],
        out_specs=pl.BlockSpec((BLK,), lambda i: (i,)),
        out_shape=jax.ShapeDtypeStruct((B,), jnp.float32),
    )(rel_mat0, rel_mat1, t_rows, a_rows)
    return out


def kernel(target_nodes, anchor_nodes, target_table, anchor_table, rel_mat0, rel_mat1):
    tgt_idx = target_nodes.astype(jnp.int32)
    anc_idx = anchor_nodes.astype(jnp.int32)
    t_rows, a_rows = _sc_gather(tgt_idx, anc_idx, target_table, anchor_table)
    return _tc_score(rel_mat0, rel_mat1, t_rows, a_rows)
